# baseline (speedup 1.0000x reference)
import jax
import jax.numpy as jnp
from jax.experimental import pallas as pl

N = 50000
V = 128
G = 256
EPS = 1e-05
BLK = 1000
NB = N // BLK


def _seg_kernel(pos_ref, batch_ref, seg_ref, cnt_ref, posf_ref):
    i = pl.program_id(0)
    x = pos_ref[...]
    x0 = x[:, 0, :]
    x1 = x[:, 1, :]
    x2 = x[:, 2, :]
    posf_ref[...] = jnp.concatenate((x0, x1, x2), axis=1)
    nrm = jnp.sqrt(x0 * x0 + x1 * x1 + x2 * x2)
    b = batch_ref[0, 0, :]
    oh = (jax.lax.broadcasted_iota(jnp.int32, (G, BLK), 0)
          == b[None, :]).astype(jnp.float32)
    part = jnp.dot(oh, nrm, preferred_element_type=jnp.float32)
    pcnt = jnp.sum(oh, axis=1)[None, :]

    @pl.when(i == 0)
    def _():
        seg_ref[...] = jnp.zeros_like(seg_ref)
        cnt_ref[...] = jnp.zeros_like(cnt_ref)

    seg_ref[...] += part
    cnt_ref[...] += pcnt


def _norm_kernel(posf_ref, batch_ref, seg_ref, cnt_ref, w_ref, out_ref):
    x = posf_ref[...]
    b = batch_ref[0, 0, :]
    cnt = jnp.maximum(cnt_ref[0, :], 1.0)
    mean = seg_ref[...] / cnt[:, None]
    oh = (b[:, None] == jax.lax.broadcasted_iota(jnp.int32, (BLK, G), 1)
          ).astype(jnp.float32)
    gm = jnp.dot(oh, mean, preferred_element_type=jnp.float32)
    w = w_ref[0, 0, :]
    scale = w[None, :] / (gm + EPS)
    out_ref[:, 0, :] = x[:, :V] * scale
    out_ref[:, 1, :] = x[:, V:2 * V] * scale
    out_ref[:, 2, :] = x[:, 2 * V:] * scale


def kernel(pos, weight, batch):
    b3 = batch.astype(jnp.int32).reshape(NB, 1, BLK)

    seg, cnt, posf = pl.pallas_call(
        _seg_kernel,
        grid=(NB,),
        in_specs=[
            pl.BlockSpec((BLK, 3, V), lambda i: (i, 0, 0)),
            pl.BlockSpec((1, 1, BLK), lambda i: (i, 0, 0)),
        ],
        out_specs=[
            pl.BlockSpec((G, V), lambda i: (0, 0)),
            pl.BlockSpec((1, G), lambda i: (0, 0)),
            pl.BlockSpec((BLK, 3 * V), lambda i: (i, 0)),
        ],
        out_shape=[
            jax.ShapeDtypeStruct((G, V), jnp.float32),
            jax.ShapeDtypeStruct((1, G), jnp.float32),
            jax.ShapeDtypeStruct((N, 3 * V), jnp.float32),
        ],
    )(pos, b3)

    out = pl.pallas_call(
        _norm_kernel,
        grid=(NB,),
        in_specs=[
            pl.BlockSpec((BLK, 3 * V), lambda i: (i, 0)),
            pl.BlockSpec((1, 1, BLK), lambda i: (i, 0, 0)),
            pl.BlockSpec((G, V), lambda i: (0, 0)),
            pl.BlockSpec((1, G), lambda i: (0, 0)),
            pl.BlockSpec((1, 1, V), lambda i: (0, 0, 0)),
        ],
        out_specs=pl.BlockSpec((BLK, 3, V), lambda i: (i, 0, 0)),
        out_shape=jax.ShapeDtypeStruct((N, 3, V), jnp.float32),
    )(posf, b3, seg, cnt, weight)

    return out


# speedup vs baseline: 1.0678x; 1.0678x over previous
import functools

import jax
import jax.numpy as jnp
from jax import lax
from jax.experimental import pallas as pl
from jax.experimental.pallas import tpu as pltpu
from jax.experimental.pallas import tpu_sc as plsc

N = 50000
V = 128
G = 256
EPS = 1e-05
BLK = 1000
NB = N // BLK

NC = 2
NS = 16
CHUNK = 128
FULL = N // CHUNK
TAIL = N - FULL * CHUNK
NCHUNK = FULL + 1
GPAD = G + 8


def _pass1_kernel(posf_ref, batch_ref, nrm_ref, cnt_ref):
    i = pl.program_id(0)
    x = posf_ref[...]
    x0 = x[:, :V]
    x1 = x[:, V:2 * V]
    x2 = x[:, 2 * V:]
    nrm_ref[...] = jnp.sqrt(x0 * x0 + x1 * x1 + x2 * x2)
    b = batch_ref[0, 0, :]
    oh = (jax.lax.broadcasted_iota(jnp.int32, (G, BLK), 0)
          == b[None, :]).astype(jnp.float32)
    pcnt = jnp.sum(oh, axis=1)[None, :]

    @pl.when(i == 0)
    def _():
        cnt_ref[...] = jnp.zeros_like(cnt_ref)

    cnt_ref[...] += pcnt


def _sc_seg_body(norm_hbm, batch_hbm, seg_hbm,
                 nrm_v, idx_v, zer_v, seg_sh):
    cid = lax.axis_index("c")
    sid = lax.axis_index("s")
    gid = cid * NS + sid

    def _zrow(r, _):
        for g in range(V // 16):
            zer_v[r, pl.ds(g * 16, 16)] = jnp.zeros((16,), jnp.float32)
        return 0
    lax.fori_loop(0, 16, _zrow, 0)
    pltpu.sync_copy(zer_v, seg_sh.at[pl.ds(sid * 16, 16)])

    @pl.when(sid == 0)
    def _():
        pltpu.sync_copy(zer_v.at[pl.ds(0, GPAD - G)],
                        seg_sh.at[pl.ds(G, GPAD - G)])

    plsc.subcore_barrier()

    for k in range(13):
        c = gid + 32 * k

        @pl.when(c < FULL)
        def _():
            base = c * CHUNK
            pltpu.sync_copy(batch_hbm.at[pl.ds(base, CHUNK)], idx_v)
            pltpu.sync_copy(norm_hbm.at[pl.ds(base, CHUNK)], nrm_v)
            pltpu.sync_copy(nrm_v, seg_sh.at[idx_v], add=True)

        @pl.when(c == FULL)
        def _():
            for m in range(CHUNK // 16):
                idx_v[pl.ds(m * 16, 16)] = jnp.full((16,), G, jnp.int32)
            pltpu.sync_copy(batch_hbm.at[pl.ds(FULL * CHUNK, TAIL)],
                            idx_v.at[pl.ds(0, TAIL)])
            pltpu.sync_copy(norm_hbm.at[pl.ds(FULL * CHUNK, TAIL)],
                            nrm_v.at[pl.ds(0, TAIL)])
            pltpu.sync_copy(nrm_v, seg_sh.at[idx_v], add=True)

    plsc.subcore_barrier()

    @pl.when(sid == 0)
    def _():
        pltpu.sync_copy(seg_sh.at[pl.ds(0, G)], seg_hbm.at[cid])


@functools.lru_cache(maxsize=1)
def _make_sc_seg():
    mesh = plsc.VectorSubcoreMesh(core_axis_name="c", subcore_axis_name="s")
    return pl.kernel(
        _sc_seg_body,
        out_type=jax.ShapeDtypeStruct((NC, G, V), jnp.float32),
        mesh=mesh,
        scratch_types=[
            pltpu.VMEM((CHUNK, V), jnp.float32),
            pltpu.VMEM((CHUNK,), jnp.int32),
            pltpu.VMEM((16, V), jnp.float32),
            pltpu.VMEM_SHARED((GPAD, V), jnp.float32),
        ],
    )


def _pass2_kernel(posf_ref, batch_ref, seg_ref, cnt_ref, w_ref, out_ref):
    x = posf_ref[...]
    b = batch_ref[0, 0, :]
    cnt = jnp.maximum(cnt_ref[0, :], 1.0)
    seg = seg_ref[0] + seg_ref[1]
    mean = seg / cnt[:, None]
    oh = (b[:, None] == jax.lax.broadcasted_iota(jnp.int32, (BLK, G), 1)
          ).astype(jnp.float32)
    gm = jnp.dot(oh, mean, preferred_element_type=jnp.float32)
    w = w_ref[0, 0, :]
    scale = w[None, :] / (gm + EPS)
    out_ref[:, :V] = x[:, :V] * scale
    out_ref[:, V:2 * V] = x[:, V:2 * V] * scale
    out_ref[:, 2 * V:] = x[:, 2 * V:] * scale


def kernel(pos, weight, batch):
    posf = pos.reshape(N, 3 * V)
    b32 = batch.astype(jnp.int32)
    b3 = b32.reshape(NB, 1, BLK)

    nrm, cnt = pl.pallas_call(
        _pass1_kernel,
        grid=(NB,),
        in_specs=[
            pl.BlockSpec((BLK, 3 * V), lambda i: (i, 0)),
            pl.BlockSpec((1, 1, BLK), lambda i: (i, 0, 0)),
        ],
        out_specs=[
            pl.BlockSpec((BLK, V), lambda i: (i, 0)),
            pl.BlockSpec((1, G), lambda i: (0, 0)),
        ],
        out_shape=[
            jax.ShapeDtypeStruct((N, V), jnp.float32),
            jax.ShapeDtypeStruct((1, G), jnp.float32),
        ],
    )(posf, b3)

    seg = _make_sc_seg()(nrm, b32)

    out = pl.pallas_call(
        _pass2_kernel,
        grid=(NB,),
        in_specs=[
            pl.BlockSpec((BLK, 3 * V), lambda i: (i, 0)),
            pl.BlockSpec((1, 1, BLK), lambda i: (i, 0, 0)),
            pl.BlockSpec((NC, G, V), lambda i: (0, 0, 0)),
            pl.BlockSpec((1, G), lambda i: (0, 0)),
            pl.BlockSpec((1, 1, V), lambda i: (0, 0, 0)),
        ],
        out_specs=pl.BlockSpec((BLK, 3 * V), lambda i: (i, 0)),
        out_shape=jax.ShapeDtypeStruct((N, 3 * V), jnp.float32),
    )(posf, b3, seg, cnt, weight)

    return out.reshape(N, 3, V)
